# baseline (speedup 1.0000x reference)
import jax
import jax.numpy as jnp
from jax import lax
from jax.experimental import pallas as pl
from jax.experimental.pallas import tpu as pltpu
from jax.experimental.pallas import tpu_sc as plsc

_BATCH = 16384
_VOCAB = 1000000
_DIM = 3
_NC, _NS = 2, 16
_NW = _NC * _NS
_CHUNK = 128
_CPW = _BATCH // (_NW * _CHUNK)
_BPW = _BATCH // _NW
_NTAB = 3
_SUB = _CHUNK // 16
_QSCALE = 4096.0


def _body(idx_hbm, w0, w1, w2, out_hbm, idx_v, fidx_v, vals_v, out_v, sem):
    c = lax.axis_index("c")
    s = lax.axis_index("s")
    w = c * _NS + s

    pltpu.sync_copy(idx_hbm.at[pl.ds(w * _BPW, _BPW)], idx_v)

    for j in range(_CPW):
        for cc in range(_SUB):
            v = idx_v[pl.ds(j * _CHUNK + cc * 16, 16)]
            lo = (v * 3) >> 1
            fidx_v[2 * j, pl.ds(cc * 16, 16)] = lo
            fidx_v[2 * j + 1, pl.ds(cc * 16, 16)] = lo + 1

    copies = []
    for t, tbl in enumerate((w0, w1, w2)):
        for r in range(2 * _CPW):
            copies.append(
                pltpu.async_copy(
                    tbl.at[fidx_v.at[r]], vals_v.at[t * 2 * _CPW + r], sem
                )
            )
    for cp in copies:
        cp.wait()

    accs = [jnp.zeros((16,), jnp.float32) for _ in range(_NTAB * _DIM)]
    for t in range(_NTAB):
        for j in range(_CPW):
            for cc in range(_SUB):
                v = idx_v[pl.ds(j * _CHUNK + cc * 16, 16)]
                peven = (v & 1) == 0
                wlo = vals_v[t * 2 * _CPW + 2 * j, pl.ds(cc * 16, 16)]
                whi = vals_v[t * 2 * _CPW + 2 * j + 1, pl.ds(cc * 16, 16)]
                lo_l = (wlo << 16) >> 16
                lo_h = wlo >> 16
                hi_l = (whi << 16) >> 16
                hi_h = whi >> 16
                d0 = jnp.where(peven, lo_l, lo_h).astype(jnp.float32)
                d1 = jnp.where(peven, lo_h, hi_l).astype(jnp.float32)
                d2 = jnp.where(peven, hi_l, hi_h).astype(jnp.float32)
                accs[t * _DIM + 0] = accs[t * _DIM + 0] + d0
                accs[t * _DIM + 1] = accs[t * _DIM + 1] + d1
                accs[t * _DIM + 2] = accs[t * _DIM + 2] + d2

    iota = lax.iota(jnp.int32, 16)
    _dnums = lax.GatherDimensionNumbers(
        offset_dims=(), collapsed_slice_dims=(0,), start_index_map=(0,)
    )

    def _shuffle(x, idx16):
        return lax.gather(
            x,
            idx16[:, None],
            _dnums,
            slice_sizes=(1,),
            mode=lax.GatherScatterMode.PROMISE_IN_BOUNDS,
        )

    def _lane_sum(x):
        for sh in (1, 2, 4, 8):
            x = x + _shuffle(x, jnp.bitwise_xor(iota, sh))
        return x

    part = jnp.zeros((16,), jnp.float32)
    for k2 in range(_NTAB * _DIM):
        part = jnp.where(iota == k2, _lane_sum(accs[k2]), part)
    scale = (jnp.where(
        iota < 3, 5.0, jnp.where(iota < 6, 10.0, jnp.where(iota < 9, 6.0, 0.0))
    ) * (1.0 / _QSCALE)).astype(jnp.float32)
    out_v[...] = part * scale

    pltpu.sync_copy(out_v, out_hbm.at[w])


_sc_call = pl.kernel(
    _body,
    out_type=jax.ShapeDtypeStruct((_NW, 16), jnp.float32),
    mesh=plsc.VectorSubcoreMesh(core_axis_name="c", subcore_axis_name="s"),
    scratch_types=[
        pltpu.VMEM((_BPW,), jnp.int32),
        pltpu.VMEM((2 * _CPW, _CHUNK), jnp.int32),
        pltpu.VMEM((_NTAB * 2 * _CPW, _CHUNK), jnp.int32),
        pltpu.VMEM((16,), jnp.float32),
        pltpu.SemaphoreType.DMA,
    ],
)


def _pack(w):
    q = jnp.clip(jnp.round(w * _QSCALE), -32768, 32767).astype(jnp.int16)
    return jax.lax.bitcast_convert_type(
        q.reshape(_VOCAB * _DIM // 2, 2), jnp.int32
    )


@jax.jit
def kernel(e_input, W0, W1, W2):
    out = _sc_call(
        e_input.astype(jnp.int32), _pack(W0), _pack(W1), _pack(W2)
    )
    return out.sum(axis=0)[: _NTAB * _DIM].reshape(_NTAB, _DIM)


# speedup vs baseline: 1.1201x; 1.1201x over previous
import jax
import jax.numpy as jnp
from jax import lax
from jax.experimental import pallas as pl
from jax.experimental.pallas import tpu as pltpu
from jax.experimental.pallas import tpu_sc as plsc

_BATCH = 16384
_VOCAB = 1000000
_DIM = 3
_NC, _NS = 2, 16
_NW = _NC * _NS
_CHUNK = 128
_CPW = _BATCH // (_NW * _CHUNK)
_BPW = _BATCH // _NW
_NTAB = 3
_SUB = _CHUNK // 16


def _body(idx_hbm, w0, w1, w2, out_hbm, idx_v, fidx_v, vals_v, out_v, sem):
    c = lax.axis_index("c")
    s = lax.axis_index("s")
    w = c * _NS + s

    pltpu.sync_copy(idx_hbm.at[pl.ds(w * _BPW, _BPW)], idx_v)

    for j in range(_CPW):
        for cc in range(_SUB):
            v3 = idx_v[pl.ds(j * _CHUNK + cc * 16, 16)] * 3
            for d in range(_DIM):
                fidx_v[d * _CPW + j, pl.ds(cc * 16, 16)] = v3 + d

    copies = []
    for t, tbl in enumerate((w0, w1, w2)):
        for dj in range(_DIM * _CPW):
            copies.append(
                pltpu.async_copy(
                    tbl.at[fidx_v.at[dj]], vals_v.at[t * _DIM * _CPW + dj], sem
                )
            )
    for cp in copies:
        cp.wait()

    accs = [jnp.zeros((16,), jnp.float32) for _ in range(_NTAB * _DIM)]
    for t in range(_NTAB):
        for d in range(_DIM):
            for j in range(_CPW):
                row = t * _DIM * _CPW + d * _CPW + j
                for cc in range(_SUB):
                    accs[t * _DIM + d] = (
                        accs[t * _DIM + d] + vals_v[row, pl.ds(cc * 16, 16)]
                    )

    iota = lax.iota(jnp.int32, 16)
    _dnums = lax.GatherDimensionNumbers(
        offset_dims=(), collapsed_slice_dims=(0,), start_index_map=(0,)
    )

    def _shuffle(v, idx16):
        return lax.gather(
            v,
            idx16[:, None],
            _dnums,
            slice_sizes=(1,),
            mode=lax.GatherScatterMode.PROMISE_IN_BOUNDS,
        )

    def _lane_sum(v):
        for sh in (1, 2, 4, 8):
            v = v + _shuffle(v, jnp.bitwise_xor(iota, sh))
        return v

    part = jnp.zeros((16,), jnp.float32)
    for k2 in range(_NTAB * _DIM):
        part = jnp.where(iota == k2, _lane_sum(accs[k2]), part)
    scale = jnp.where(
        iota < 3, 5.0, jnp.where(iota < 6, 10.0, jnp.where(iota < 9, 6.0, 0.0))
    ).astype(jnp.float32)
    out_v[...] = part * scale

    pltpu.sync_copy(out_v, out_hbm.at[w])


_sc_call = pl.kernel(
    _body,
    out_type=jax.ShapeDtypeStruct((_NW, 16), jnp.float32),
    mesh=plsc.VectorSubcoreMesh(core_axis_name="c", subcore_axis_name="s"),
    scratch_types=[
        pltpu.VMEM((_BPW,), jnp.int32),
        pltpu.VMEM((_DIM * _CPW, _CHUNK), jnp.int32),
        pltpu.VMEM((_NTAB * _DIM * _CPW, _CHUNK), jnp.float32),
        pltpu.VMEM((16,), jnp.float32),
        pltpu.SemaphoreType.DMA,
    ],
)


@jax.jit
def kernel(e_input, W0, W1, W2):
    out = _sc_call(
        e_input.astype(jnp.int32),
        W0.reshape(_VOCAB * _DIM),
        W1.reshape(_VOCAB * _DIM),
        W2.reshape(_VOCAB * _DIM),
    )
    return out.sum(axis=0)[: _NTAB * _DIM].reshape(_NTAB, _DIM)


# speedup vs baseline: 78.3688x; 69.9642x over previous
import jax
import jax.numpy as jnp
from jax import lax
from jax.experimental import pallas as pl
from jax.experimental.pallas import tpu as pltpu
from jax.experimental.pallas import tpu_sc as plsc

_BATCH = 16384
_VOCAB = 1000000
_DIM = 3
_NC, _NS = 2, 16
_NW = _NC * _NS
_CHUNK = 128
_CPW = _BATCH // (_NW * _CHUNK)
_BPW = _BATCH // _NW
_NTAB = 3
_SUB = _CHUNK // 16


def _body(idx_hbm, w0, w1, w2, out_hbm, idx_v, fidx_v, vals_v, out_v, sem):
    c = lax.axis_index("c")
    s = lax.axis_index("s")
    w = c * _NS + s

    pltpu.sync_copy(idx_hbm.at[pl.ds(w * _BPW, _BPW)], idx_v)

    for j in range(_CPW):
        for cc in range(_SUB):
            v = idx_v[pl.ds(j * _CHUNK + cc * 16, 16)]
            for d in range(_DIM):
                fidx_v[d * _CPW + j, pl.ds(cc * 16, 16)] = v + d * _VOCAB

    copies = []
    for t, tbl in enumerate((w0, w1, w2)):
        for dj in range(_DIM * _CPW):
            copies.append(
                pltpu.async_copy(
                    tbl.at[fidx_v.at[dj]], vals_v.at[t * _DIM * _CPW + dj], sem
                )
            )
    for cp in copies:
        cp.wait()

    accs = [jnp.zeros((16,), jnp.float32) for _ in range(_NTAB * _DIM)]
    for t in range(_NTAB):
        for d in range(_DIM):
            for j in range(_CPW):
                row = t * _DIM * _CPW + d * _CPW + j
                for cc in range(_SUB):
                    accs[t * _DIM + d] = (
                        accs[t * _DIM + d] + vals_v[row, pl.ds(cc * 16, 16)]
                    )

    iota = lax.iota(jnp.int32, 16)
    _dnums = lax.GatherDimensionNumbers(
        offset_dims=(), collapsed_slice_dims=(0,), start_index_map=(0,)
    )

    def _shuffle(v, idx16):
        return lax.gather(
            v,
            idx16[:, None],
            _dnums,
            slice_sizes=(1,),
            mode=lax.GatherScatterMode.PROMISE_IN_BOUNDS,
        )

    def _lane_sum(v):
        for sh in (1, 2, 4, 8):
            v = v + _shuffle(v, jnp.bitwise_xor(iota, sh))
        return v

    part = jnp.zeros((16,), jnp.float32)
    for k2 in range(_NTAB * _DIM):
        part = jnp.where(iota == k2, _lane_sum(accs[k2]), part)
    scale = jnp.where(
        iota < 3, 5.0, jnp.where(iota < 6, 10.0, jnp.where(iota < 9, 6.0, 0.0))
    ).astype(jnp.float32)
    out_v[...] = part * scale

    pltpu.sync_copy(out_v, out_hbm.at[w])


_sc_call = pl.kernel(
    _body,
    out_type=jax.ShapeDtypeStruct((_NW, 16), jnp.float32),
    mesh=plsc.VectorSubcoreMesh(core_axis_name="c", subcore_axis_name="s"),
    scratch_types=[
        pltpu.VMEM((_BPW,), jnp.int32),
        pltpu.VMEM((_DIM * _CPW, _CHUNK), jnp.int32),
        pltpu.VMEM((_NTAB * _DIM * _CPW, _CHUNK), jnp.float32),
        pltpu.VMEM((16,), jnp.float32),
        pltpu.SemaphoreType.DMA,
    ],
)


@jax.jit
def kernel(e_input, W0, W1, W2):
    out = _sc_call(
        e_input.astype(jnp.int32),
        W0.T.reshape(_DIM * _VOCAB),
        W1.T.reshape(_DIM * _VOCAB),
        W2.T.reshape(_DIM * _VOCAB),
    )
    return out.sum(axis=0)[: _NTAB * _DIM].reshape(_NTAB, _DIM)
